# baseline (speedup 1.0000x reference)
import functools

import jax
import jax.numpy as jnp
from jax import lax
from jax.experimental import pallas as pl
from jax.experimental.pallas import tpu as pltpu
from jax.experimental.pallas import tpu_sc as plsc

N = 10000
NP = 10240
E = 320000
H = 128
NC = 2
NS = 16
NW = NC * NS
ER = 160
EC = 64
EP = NW * ER * EC
EPR = EP // EC
NPW = NP // NS
FAST_C = 0
ROWS_F = 160
ROWS_S = 160


def _k1_body(x_ref, w1_ref, b1_ref, a1_ref, bc_ref, sens_ref, h_ref, eq_ref,
             eq1_ref):
    h = jnp.dot(x_ref[...], w1_ref[...], preferred_element_type=jnp.float32)
    h = h + b1_ref[...]
    q = jnp.dot(h, a1_ref[...], preferred_element_type=jnp.float32)
    q = q - bc_ref[0, 0] * sens_ref[...]
    m = jnp.max(q)
    eq = jnp.exp(q - m)
    h_ref[:N, :] = h
    h_ref[N:, :] = jnp.zeros((NP - N, H), jnp.float32)
    eq_ref[:N, :] = eq
    eq_ref[N:, :] = jnp.zeros((NP - N, 1), jnp.float32)
    eqz = jnp.concatenate([eq, jnp.zeros((NP - N, 1), jnp.float32)], axis=0)
    eq1_ref[...] = eqz.reshape(NP)


_k1 = pl.pallas_call(
    _k1_body,
    out_shape=(
        jax.ShapeDtypeStruct((NP, H), jnp.float32),
        jax.ShapeDtypeStruct((NP, 1), jnp.float32),
        jax.ShapeDtypeStruct((NP,), jnp.float32),
    ),
)


def _k2_body(eq_hbm, src_hbm, dst_hbm, out_hbm, src_t, dst_t, eq_t, t0_t):
    c = lax.axis_index("c")
    s = lax.axis_index("s")
    wid = c * NS + s
    pltpu.sync_copy(src_hbm.at[pl.ds(wid * ER, ER)], src_t)
    pltpu.sync_copy(dst_hbm.at[pl.ds(wid * ER, ER)], dst_t)
    pltpu.sync_copy(eq_hbm, eq_t)

    def _zero(i, carry):
        t0_t[pl.ds(i * 16, 16)] = jnp.zeros((16,), jnp.float32)
        return carry

    lax.fori_loop(0, NP // 16, _zero, 0)

    def _edge(r, carry):
        for k in range(EC // 16):
            d = dst_t[r, pl.ds(k * 16, 16)]
            e = plsc.load_gather(eq_t, [d])
            srow = src_t[r, pl.ds(k * 16, 16)]
            plsc.addupdate_scatter(t0_t, [srow], e)
        return carry


    lax.fori_loop(0, ER, _edge, 0)
    pltpu.sync_copy(t0_t, out_hbm.at[wid, 0])


_k2 = pl.kernel(
    _k2_body,
    out_type=jax.ShapeDtypeStruct((NW, 1, NP), jnp.float32),
    mesh=plsc.VectorSubcoreMesh(core_axis_name="c", subcore_axis_name="s"),
    compiler_params=pltpu.CompilerParams(needs_layout_passes=False),
    scratch_types=[
        pltpu.VMEM((ER, EC), jnp.int32),
        pltpu.VMEM((ER, EC), jnp.int32),
        pltpu.VMEM((NP,), jnp.float32),
        pltpu.VMEM((NP,), jnp.float32),
    ],
)


def _k3_body(h_ref, eq_ref, t0p_ref, v_ref):
    t0 = jnp.sum(t0p_ref[...], axis=(0, 1), keepdims=False)[None, :]
    denom = t0.T + eq_ref[...] + 1e-16
    v_ref[...] = h_ref[...] / denom


_k3 = pl.pallas_call(
    _k3_body,
    out_shape=jax.ShapeDtypeStruct((NP, H), jnp.float32),
)


def _k4_body(v_hbm, src_hbm, dst_hbm, out_hbm, sidx, didx, isems, rbufs,
             gsems, ssems, acc):
    c = lax.axis_index("c")
    s = lax.axis_index("s")

    def _zrow(i, carry):
        for k in range(H // 16):
            rbufs[0][i, pl.ds(k * 16, 16)] = jnp.zeros((16,), jnp.float32)
        return carry

    lax.fori_loop(0, EC, _zrow, 0)
    for t in range(NPW // EC):
        pltpu.sync_copy(rbufs[0], acc.at[pl.ds(s * NPW + t * EC, EC)])
    plsc.subcore_barrier()

    def _stage(row0, par):
        pltpu.async_copy(src_hbm.at[pl.ds(row0, 8)], sidx[par],
                         isems[2 * par])
        pltpu.async_copy(dst_hbm.at[pl.ds(row0, 8)], didx[par],
                         isems[2 * par + 1])

    def _stage_wait(par):
        pltpu.make_async_copy(src_hbm.at[pl.ds(0, 8)], sidx[par],
                              isems[2 * par]).wait()
        pltpu.make_async_copy(dst_hbm.at[pl.ds(0, 8)], didx[par],
                              isems[2 * par + 1]).wait()

    def _step(base, k, first=False, last=False):
        slot = k % 4
        nslot = (k + 3) % 4
        p_cur = (k // 8) % 2
        pltpu.make_async_copy(v_hbm.at[sidx[p_cur].at[k % 8]], rbufs[slot],
                              gsems[slot]).wait()
        if not (first and k == 0):
            pltpu.make_async_copy(rbufs[nslot], acc.at[didx[p_cur].at[k % 8]],
                                  ssems[nslot]).wait()
        if k == 1:
            _stage(base + 8, 1)
        if k == 5:
            _stage_wait(1)
        if k == 8 and not last:
            _stage(base + 16, 0)
        if k == 13 and not last:
            _stage_wait(0)
        if not (last and k >= 13):
            gp = ((k + 3) // 8) % 2
            pltpu.async_copy(v_hbm.at[sidx[gp].at[(k + 3) % 8]], rbufs[nslot],
                             gsems[nslot])
        pltpu.async_copy(rbufs[slot], acc.at[didx[p_cur].at[k % 8]],
                         ssems[slot], add=True)

    is_fast = c == FAST_C
    row0 = jnp.where(is_fast, s * ROWS_F, NS * ROWS_F + s * ROWS_S)
    npair = jnp.where(is_fast, ROWS_F // 16, ROWS_S // 16)

    pltpu.sync_copy(src_hbm.at[pl.ds(row0, 8)], sidx[0])
    pltpu.sync_copy(dst_hbm.at[pl.ds(row0, 8)], didx[0])
    for r in range(3):
        pltpu.async_copy(v_hbm.at[sidx[0].at[r]], rbufs[r], gsems[r])

    for k in range(16):
        _step(row0, k, first=True)

    def _pair(g, carry):
        for k in range(16):
            _step(row0 + g * 16, k)
        return carry

    lax.fori_loop(1, npair - 1, _pair, 0)

    for k in range(16):
        _step(row0 + (npair - 1) * 16, k, last=True)
    pltpu.make_async_copy(rbufs[3], acc.at[didx[1].at[7]], ssems[3]).wait()

    plsc.subcore_barrier()

    for t in range(NPW // EC):
        rows = pl.ds(s * NPW + t * EC, EC)
        pltpu.sync_copy(acc.at[rows], rbufs[t % 4])
        pltpu.sync_copy(rbufs[t % 4], out_hbm.at[c].at[rows])


_k4 = pl.kernel(
    _k4_body,
    out_type=jax.ShapeDtypeStruct((NC, NP, H), jnp.float32),
    mesh=plsc.VectorSubcoreMesh(core_axis_name="c", subcore_axis_name="s"),
    compiler_params=pltpu.CompilerParams(needs_layout_passes=False),
    scratch_types=[
        [pltpu.VMEM((8, EC), jnp.int32) for _ in range(2)],
        [pltpu.VMEM((8, EC), jnp.int32) for _ in range(2)],
        [pltpu.SemaphoreType.DMA for _ in range(4)],
        [pltpu.VMEM((EC, H), jnp.float32) for _ in range(4)],
        [pltpu.SemaphoreType.DMA for _ in range(4)],
        [pltpu.SemaphoreType.DMA for _ in range(4)],
        pltpu.VMEM_SHARED((NP, H), jnp.float32),
    ],
)


def _k5_body(v_ref, eq_ref, p_ref, fcw_ref, fcb_ref, o_ref):
    srow = v_ref[...] + p_ref[0] + p_ref[1]
    out = eq_ref[...] * srow
    u = jnp.maximum(out, 0.0)
    logits = jnp.dot(u, fcw_ref[...], preferred_element_type=jnp.float32)
    logits = logits + fcb_ref[...]
    m = jnp.max(logits, axis=1, keepdims=True)
    lse = m + jnp.log(jnp.sum(jnp.exp(logits - m), axis=1, keepdims=True))
    res = logits - lse
    o_ref[...] = res[:N, :]


_k5 = pl.pallas_call(
    _k5_body,
    out_shape=jax.ShapeDtypeStruct((N, 2), jnp.float32),
)


def kernel(x, edge_index, W1, b1, att_W, att_b, bias_correction, sens, fc_W,
           fc_b):
    pad = (N + jnp.arange(EP - E, dtype=jnp.int32) % (NP - N)).reshape(-1, EC)
    ei3 = edge_index.reshape(2, E // EC, EC)
    src = jnp.concatenate([ei3[0], pad], axis=0)
    dst = jnp.concatenate([ei3[1], pad], axis=0)
    a1 = att_W[:H, :]
    bc = bias_correction.reshape(1, 1)
    sens_c = sens.reshape(N, 1)

    h, eq, eq1 = _k1(x, W1, b1.reshape(1, H), a1, bc, sens_c)
    t0p = _k2(eq1, src, dst)
    v = _k3(h, eq, t0p)
    parts = _k4(v, src, dst)
    return _k5(v, eq, parts, fc_W, fc_b.reshape(1, 2))


# speedup vs baseline: 1.0445x; 1.0445x over previous
import functools

import jax
import jax.numpy as jnp
from jax import lax
from jax.experimental import pallas as pl
from jax.experimental.pallas import tpu as pltpu
from jax.experimental.pallas import tpu_sc as plsc

N = 10000
NP = 10240
E = 320000
H = 128
NC = 2
NS = 16
NW = NC * NS
ER = 160
EC = 64
EP = NW * ER * EC
EPR = EP // EC
NPW = NP // NS
FAST_C = 0
ROWS_F = 160
ROWS_S = 160


def _k1_body(x_ref, w1_ref, b1_ref, a1_ref, bc_ref, sens_ref, h_ref, eq_ref,
             eq1_ref):
    h = jnp.dot(x_ref[...], w1_ref[...], preferred_element_type=jnp.float32)
    h = h + b1_ref[...]
    q = jnp.dot(h, a1_ref[...], preferred_element_type=jnp.float32)
    q = q - bc_ref[0, 0] * sens_ref[...]
    m = jnp.max(q)
    eq = jnp.exp(q - m)
    h_ref[:N, :] = h
    h_ref[N:, :] = jnp.zeros((NP - N, H), jnp.float32)
    eq_ref[:N, :] = eq
    eq_ref[N:, :] = jnp.zeros((NP - N, 1), jnp.float32)
    eqz = jnp.concatenate([eq, jnp.zeros((NP - N, 1), jnp.float32)], axis=0)
    eq1_ref[...] = eqz.reshape(NP)


_k1 = pl.pallas_call(
    _k1_body,
    out_shape=(
        jax.ShapeDtypeStruct((NP, H), jnp.float32),
        jax.ShapeDtypeStruct((NP, 1), jnp.float32),
        jax.ShapeDtypeStruct((NP,), jnp.float32),
    ),
)


def _k2_body(eq_hbm, src_hbm, dst_hbm, out_hbm, src_t, dst_t, eq_t, t0_t):
    c = lax.axis_index("c")
    s = lax.axis_index("s")
    wid = c * NS + s
    pltpu.sync_copy(src_hbm.at[pl.ds(wid * ER, ER)], src_t)
    pltpu.sync_copy(dst_hbm.at[pl.ds(wid * ER, ER)], dst_t)
    pltpu.sync_copy(eq_hbm, eq_t)

    def _zero(i, carry):
        t0_t[pl.ds(i * 16, 16)] = jnp.zeros((16,), jnp.float32)
        return carry

    lax.fori_loop(0, NP // 16, _zero, 0)

    def _edge(r, carry):
        for k in range(EC // 16):
            d = dst_t[r, pl.ds(k * 16, 16)]
            e = plsc.load_gather(eq_t, [d])
            srow = src_t[r, pl.ds(k * 16, 16)]
            plsc.addupdate_scatter(t0_t, [srow], e)
        return carry


    lax.fori_loop(0, ER, _edge, 0)
    pltpu.sync_copy(t0_t, out_hbm.at[wid, 0])


_k2 = pl.kernel(
    _k2_body,
    out_type=jax.ShapeDtypeStruct((NW, 1, NP), jnp.float32),
    mesh=plsc.VectorSubcoreMesh(core_axis_name="c", subcore_axis_name="s"),
    compiler_params=pltpu.CompilerParams(needs_layout_passes=False),
    scratch_types=[
        pltpu.VMEM((ER, EC), jnp.int32),
        pltpu.VMEM((ER, EC), jnp.int32),
        pltpu.VMEM((NP,), jnp.float32),
        pltpu.VMEM((NP,), jnp.float32),
    ],
)


def _k3_body(h_ref, eq_ref, t0p_ref, v_ref):
    t0 = jnp.sum(t0p_ref[...], axis=(0, 1), keepdims=False)[None, :]
    denom = t0.T + eq_ref[...] + 1e-16
    v_ref[...] = h_ref[...] / denom


_k3 = pl.pallas_call(
    _k3_body,
    out_shape=jax.ShapeDtypeStruct((NP, H), jnp.float32),
)


def _k4_body(v_hbm, src_hbm, dst_hbm, out_hbm, sidx, didx, isems, rbufs,
             gsems, ssems, acc):
    c = lax.axis_index("c")
    s = lax.axis_index("s")

    def _zrow(i, carry):
        for k in range(H // 16):
            rbufs[0][i, pl.ds(k * 16, 16)] = jnp.zeros((16,), jnp.float32)
        return carry

    lax.fori_loop(0, EC, _zrow, 0)
    for t in range(NPW // EC):
        pltpu.sync_copy(rbufs[0], acc.at[pl.ds(s * NPW + t * EC, EC)])
    plsc.subcore_barrier()

    def _stage(row0, par):
        pltpu.async_copy(src_hbm.at[pl.ds(row0, 8)], sidx[par],
                         isems[2 * par])
        pltpu.async_copy(dst_hbm.at[pl.ds(row0, 8)], didx[par],
                         isems[2 * par + 1])

    def _stage_wait(par):
        pltpu.make_async_copy(src_hbm.at[pl.ds(0, 8)], sidx[par],
                              isems[2 * par]).wait()
        pltpu.make_async_copy(dst_hbm.at[pl.ds(0, 8)], didx[par],
                              isems[2 * par + 1]).wait()

    def _step(base, k, first=False, last=False):
        slot = k % 4
        nslot = (k + 3) % 4
        p_cur = (k // 8) % 2
        pltpu.make_async_copy(v_hbm.at[sidx[p_cur].at[k % 8]], rbufs[slot],
                              gsems[slot]).wait()
        if not (first and k == 0):
            pltpu.make_async_copy(rbufs[nslot], acc.at[didx[p_cur].at[k % 8]],
                                  ssems[nslot]).wait()
        if k == 1:
            _stage(base + 8, 1)
        if k == 5:
            _stage_wait(1)
        if k == 8 and not last:
            _stage(base + 16, 0)
        if k == 13 and not last:
            _stage_wait(0)
        if not (last and k >= 13):
            gp = ((k + 3) // 8) % 2
            pltpu.async_copy(v_hbm.at[sidx[gp].at[(k + 3) % 8]], rbufs[nslot],
                             gsems[nslot])
        pltpu.async_copy(rbufs[slot], acc.at[didx[p_cur].at[k % 8]],
                         ssems[slot], add=True)

    is_fast = c == FAST_C
    row0 = jnp.where(is_fast, s * ROWS_F, NS * ROWS_F + s * ROWS_S)
    npair = jnp.where(is_fast, ROWS_F // 16, ROWS_S // 16)

    pltpu.sync_copy(src_hbm.at[pl.ds(row0, 8)], sidx[0])
    pltpu.sync_copy(dst_hbm.at[pl.ds(row0, 8)], didx[0])
    for r in range(3):
        pltpu.async_copy(v_hbm.at[sidx[0].at[r]], rbufs[r], gsems[r])

    for k in range(16):
        _step(row0, k, first=True)

    def _pair(g, carry):
        for k in range(16):
            _step(row0 + g * 16, k)
        return carry

    lax.fori_loop(1, npair - 1, _pair, 0)

    for k in range(16):
        _step(row0 + (npair - 1) * 16, k, last=True)
    pltpu.make_async_copy(rbufs[3], acc.at[didx[1].at[7]], ssems[3]).wait()

    plsc.subcore_barrier()

    for t in range(NPW // EC):
        rows = pl.ds(s * NPW + t * EC, EC)
        pltpu.sync_copy(acc.at[rows], rbufs[t % 4])
        pltpu.sync_copy(rbufs[t % 4], out_hbm.at[c].at[rows])


_k4 = pl.kernel(
    _k4_body,
    out_type=jax.ShapeDtypeStruct((NC, NP, H), jnp.float32),
    mesh=plsc.VectorSubcoreMesh(core_axis_name="c", subcore_axis_name="s"),
    compiler_params=pltpu.CompilerParams(needs_layout_passes=False),
    scratch_types=[
        [pltpu.VMEM((8, EC), jnp.int32) for _ in range(2)],
        [pltpu.VMEM((8, EC), jnp.int32) for _ in range(2)],
        [pltpu.SemaphoreType.DMA for _ in range(4)],
        [pltpu.VMEM((EC, H), jnp.float32) for _ in range(4)],
        [pltpu.SemaphoreType.DMA for _ in range(4)],
        [pltpu.SemaphoreType.DMA for _ in range(4)],
        pltpu.VMEM_SHARED((NP, H), jnp.float32),
    ],
)


def _k5_body(v_ref, eq_ref, p_ref, fcw_ref, fcb_ref, o_ref):
    srow = v_ref[...] + p_ref[0] + p_ref[1]
    out = eq_ref[...] * srow
    u = jnp.maximum(out, 0.0)
    logits = jnp.dot(u, fcw_ref[...], preferred_element_type=jnp.float32)
    logits = logits + fcb_ref[...]
    m = jnp.max(logits, axis=1, keepdims=True)
    lse = m + jnp.log(jnp.sum(jnp.exp(logits - m), axis=1, keepdims=True))
    res = logits - lse
    o_ref[...] = res[:N, :]


_k5 = pl.pallas_call(
    _k5_body,
    out_shape=jax.ShapeDtypeStruct((N, 2), jnp.float32),
)


def kernel(x, edge_index, W1, b1, att_W, att_b, bias_correction, sens, fc_W,
           fc_b):
    pad = N + jnp.arange(EP - E, dtype=jnp.int32) % (NP - N)
    src = jnp.concatenate([edge_index[0], pad]).reshape(EPR, EC)
    dst = jnp.concatenate([edge_index[1], pad]).reshape(EPR, EC)
    a1 = att_W[:H, :]
    bc = bias_correction.reshape(1, 1)
    sens_c = sens.reshape(N, 1)

    h, eq, eq1 = _k1(x, W1, b1.reshape(1, H), a1, bc, sens_c)
    t0p = _k2(eq1, src, dst)
    v = _k3(h, eq, t0p)
    parts = _k4(v, src, dst)
    return _k5(v, eq, parts, fc_W, fc_b.reshape(1, 2))
